# baseline (speedup 1.0000x reference)
import functools

import jax
import jax.numpy as jnp
from jax import lax
from jax.experimental import pallas as pl
from jax.experimental.pallas import tpu as pltpu
from jax.experimental.pallas import tpu_sc as plsc


def kernel(user, item, user_table, item_table):
    B = user.shape[0]
    D = user_table.shape[1]
    info = plsc.get_sparse_core_info()
    NC, NS = info.num_cores, info.num_subcores
    NW = NC * NS
    assert B % (8 * NW) == 0
    b_per_w = B // NW
    NCHUNK = 4
    CH = b_per_w // NCHUNK

    mesh = plsc.VectorSubcoreMesh(core_axis_name="c", subcore_axis_name="s")

    @functools.partial(
        pl.kernel,
        mesh=mesh,
        out_type=(
            jax.ShapeDtypeStruct((B, D), jnp.float32),
            jax.ShapeDtypeStruct((B, D), jnp.float32),
        ),
        scratch_types=[
            pltpu.VMEM((b_per_w,), jnp.int32),
            pltpu.VMEM((b_per_w,), jnp.int32),
            pltpu.VMEM((CH, D), jnp.float32),
            pltpu.VMEM((CH, D), jnp.float32),
            pltpu.SemaphoreType.DMA,
            pltpu.SemaphoreType.DMA,
            pltpu.SemaphoreType.DMA,
            pltpu.SemaphoreType.DMA,
        ],
    )
    def gather2(user_hbm, item_hbm, ut_hbm, it_hbm, uout_hbm, iout_hbm,
                uidx_v, iidx_v, buf_a, buf_b, sga, sgb, swa, swb):
        wid = lax.axis_index("s") * NC + lax.axis_index("c")
        base = wid * b_per_w
        pltpu.sync_copy(user_hbm.at[pl.ds(base, b_per_w)], uidx_v)
        pltpu.sync_copy(item_hbm.at[pl.ds(base, b_per_w)], iidx_v)
        bufs = (buf_a, buf_b)
        gsems = (sga, sgb)
        wsems = (swa, swb)
        seq = []
        for c in range(NCHUNK):
            seq.append((ut_hbm, uidx_v, uout_hbm, c * CH))
            seq.append((it_hbm, iidx_v, iout_hbm, c * CH))
        pending = [None, None]
        for k, (tbl, idxr, outr, off) in enumerate(seq):
            slot = k % 2
            if pending[slot] is not None:
                pending[slot].wait()
            g = pltpu.async_copy(
                tbl.at[idxr.at[pl.ds(off, CH)]], bufs[slot], gsems[slot])
            g.wait()
            pending[slot] = pltpu.async_copy(
                bufs[slot], outr.at[pl.ds(base + off, CH)], wsems[slot])
        pending[0].wait()
        pending[1].wait()

    return gather2(user, item, user_table, item_table)


# speedup vs baseline: 1.0776x; 1.0776x over previous
import functools

import jax
import jax.numpy as jnp
from jax import lax
from jax.experimental import pallas as pl
from jax.experimental.pallas import tpu as pltpu
from jax.experimental.pallas import tpu_sc as plsc


def kernel(user, item, user_table, item_table):
    B = user.shape[0]
    D = user_table.shape[1]
    info = plsc.get_sparse_core_info()
    NC, NS = info.num_cores, info.num_subcores
    NW = NC * NS
    assert B % (8 * NW) == 0
    b_per_w = B // NW
    CH2 = 448
    CH3 = b_per_w - CH2

    mesh = plsc.VectorSubcoreMesh(core_axis_name="c", subcore_axis_name="s")

    @functools.partial(
        pl.kernel,
        mesh=mesh,
        out_type=(
            jax.ShapeDtypeStruct((B, D), jnp.float32),
            jax.ShapeDtypeStruct((B, D), jnp.float32),
        ),
        scratch_types=[
            pltpu.VMEM((b_per_w,), jnp.int32),
            pltpu.VMEM((b_per_w,), jnp.int32),
            pltpu.VMEM((b_per_w, D), jnp.float32),
            pltpu.VMEM((CH2, D), jnp.float32),
            pltpu.SemaphoreType.DMA,
            pltpu.SemaphoreType.DMA,
            pltpu.SemaphoreType.DMA,
            pltpu.SemaphoreType.DMA,
        ],
    )
    def gather2(user_hbm, item_hbm, ut_hbm, it_hbm, uout_hbm, iout_hbm,
                uidx_v, iidx_v, buf1, buf2, si1, si2, sg, sw):
        wid = lax.axis_index("s") * NC + lax.axis_index("c")
        base = wid * b_per_w
        c_u = pltpu.async_copy(user_hbm.at[pl.ds(base, b_per_w)], uidx_v, si1)
        c_i = pltpu.async_copy(item_hbm.at[pl.ds(base, b_per_w)], iidx_v, si2)
        c_u.wait()
        pltpu.async_copy(ut_hbm.at[uidx_v], buf1, sg).wait()
        w_u = pltpu.async_copy(buf1, uout_hbm.at[pl.ds(base, b_per_w)], sw)
        c_i.wait()
        pltpu.async_copy(it_hbm.at[iidx_v.at[pl.ds(0, CH2)]], buf2, sg).wait()
        w_u.wait()
        pltpu.async_copy(
            it_hbm.at[iidx_v.at[pl.ds(CH2, CH3)]],
            buf1.at[pl.ds(0, CH3)], sg).wait()
        w_i = pltpu.async_copy(buf2, iout_hbm.at[pl.ds(base, CH2)], sw)
        pltpu.sync_copy(buf1.at[pl.ds(0, CH3)],
                        iout_hbm.at[pl.ds(base + CH2, CH3)])
        w_i.wait()

    return gather2(user, item, user_table, item_table)


# speedup vs baseline: 1.1021x; 1.0227x over previous
import functools

import jax
import jax.numpy as jnp
from jax import lax
from jax.experimental import pallas as pl
from jax.experimental.pallas import tpu as pltpu
from jax.experimental.pallas import tpu_sc as plsc


def kernel(user, item, user_table, item_table):
    B = user.shape[0]
    D = user_table.shape[1]
    info = plsc.get_sparse_core_info()
    NC, NS = info.num_cores, info.num_subcores
    NW = NC * NS
    assert B % (8 * NW) == 0
    b_per_w = B // NW

    mesh = plsc.VectorSubcoreMesh(core_axis_name="c", subcore_axis_name="s")

    @functools.partial(
        pl.kernel,
        mesh=mesh,
        out_type=(
            jax.ShapeDtypeStruct((B, D), jnp.float32),
            jax.ShapeDtypeStruct((B, D), jnp.float32),
        ),
        scratch_types=[
            pltpu.VMEM((b_per_w,), jnp.int32),
            pltpu.VMEM((b_per_w,), jnp.int32),
            pltpu.VMEM((b_per_w, D), jnp.float32),
            pltpu.SemaphoreType.DMA,
            pltpu.SemaphoreType.DMA,
            pltpu.SemaphoreType.DMA,
        ],
    )
    def gather2(user_hbm, item_hbm, ut_hbm, it_hbm, uout_hbm, iout_hbm,
                uidx_v, iidx_v, rows_v, si1, si2, sg):
        wid = lax.axis_index("s") * NC + lax.axis_index("c")
        base = wid * b_per_w
        c_u = pltpu.async_copy(user_hbm.at[pl.ds(base, b_per_w)], uidx_v, si1)
        c_i = pltpu.async_copy(item_hbm.at[pl.ds(base, b_per_w)], iidx_v, si2)
        c_u.wait()
        pltpu.async_copy(ut_hbm.at[uidx_v], rows_v, sg).wait()
        pltpu.sync_copy(rows_v, uout_hbm.at[pl.ds(base, b_per_w)])
        c_i.wait()
        pltpu.async_copy(it_hbm.at[iidx_v], rows_v, sg).wait()
        pltpu.sync_copy(rows_v, iout_hbm.at[pl.ds(base, b_per_w)])

    return gather2(user, item, user_table, item_table)
